# Initial kernel scaffold; baseline (speedup 1.0000x reference)
import functools

import jax
import jax.numpy as jnp
from jax import lax
from jax.experimental import pallas as pl
from jax.experimental.pallas import tpu as pltpu
from jax.experimental.pallas import tpu_sc as plsc

N_USERS = 50000
N_ITEMS = 50000
NN = N_USERS + N_ITEMS
EMB = 32
E = 1600000

NC = 2
NS = 16
HALF = NN // NC
TRASH = 16 * NS
ACC_ROWS = HALF + TRASH

KSTEP = 16
CHUNK = KSTEP * 128
NCHUNK = 49
EPT = NCHUNK * CHUNK
EPAD = EPT * NS
ZROWS = 64
ACC_PER_TILE = ACC_ROWS // NS
OUT_PER_TILE = HALF // NS

_mesh = plsc.VectorSubcoreMesh(
    core_axis_name="c", subcore_axis_name="s", num_cores=NC, num_subcores=NS)


def _side_body(src_hbm, dst_hbm, w_hbm, ego_hbm, out_hbm,
               src_v, dst_v, w_v, idx_v, rows_v, zbuf, acc, sem):
  cid = lax.axis_index("c")
  sid = lax.axis_index("s")
  base = cid * HALF

  def _zero_buf(i, _):
    zbuf[i, pl.ds(0, 16)] = jnp.zeros((16,), jnp.float32)
    zbuf[i, pl.ds(16, 16)] = jnp.zeros((16,), jnp.float32)
    return 0
  lax.fori_loop(0, ZROWS, _zero_buf, 0)

  def _zero_acc(i, _):
    pltpu.sync_copy(zbuf, acc.at[pl.ds(sid * ACC_PER_TILE + i * ZROWS, ZROWS)])
    return 0
  lax.fori_loop(0, ACC_PER_TILE // ZROWS, _zero_acc, 0)
  plsc.subcore_barrier()

  trash0 = HALF + sid * 16
  lane8 = lax.iota(jnp.int32, 16) & 7

  def _chunk(ci, _):
    off = sid * EPT + ci * CHUNK
    pltpu.sync_copy(src_hbm.at[pl.ds(off, CHUNK)], src_v)
    pltpu.sync_copy(dst_hbm.at[pl.ds(off, CHUNK)], dst_v)
    pltpu.sync_copy(w_hbm.at[pl.ds(off, CHUNK)], w_v)
    copies = [
        pltpu.async_copy(ego_hbm.at[src_v.at[j]],
                         rows_v.at[pl.ds(j * 128, 128)], sem)
        for j in range(KSTEP)
    ]
    for c in copies:
      c.wait()

    def _blk(b, _):
      r0 = b * 16
      d = dst_v[pl.ds(r0, 16)]
      loc = d - base
      ok = (loc >= 0) & (loc < HALF)
      idx_v[pl.ds(r0, 16)] = jnp.where(ok, loc, trash0 + lane8)
      for j in range(16):
        r = r0 + j
        ws = w_v[r]
        rows_v[r, pl.ds(0, 16)] = rows_v[r, pl.ds(0, 16)] * ws
        rows_v[r, pl.ds(16, 16)] = rows_v[r, pl.ds(16, 16)] * ws
      return 0
    lax.fori_loop(0, CHUNK // 16, _blk, 0)

    for j in range(KSTEP):
      pltpu.sync_copy(rows_v.at[pl.ds(j * 128, 128)],
                      acc.at[idx_v.at[j]], add=True)
    return 0
  lax.fori_loop(0, NCHUNK, _chunk, 0)
  plsc.subcore_barrier()

  pltpu.sync_copy(
      acc.at[pl.ds(sid * OUT_PER_TILE, OUT_PER_TILE)],
      out_hbm.at[pl.ds(base + sid * OUT_PER_TILE, OUT_PER_TILE)])


_side_kernel = pl.kernel(
    _side_body,
    out_type=jax.ShapeDtypeStruct((NN, EMB), jnp.float32),
    mesh=_mesh,
    scratch_types=[
        pltpu.VMEM((KSTEP, 128), jnp.int32),
        pltpu.VMEM((CHUNK,), jnp.int32),
        pltpu.VMEM((CHUNK,), jnp.float32),
        pltpu.VMEM((KSTEP, 128), jnp.int32),
        pltpu.VMEM((CHUNK, EMB), jnp.float32),
        pltpu.VMEM((ZROWS, EMB), jnp.float32),
        pltpu.VMEM_SHARED((ACC_ROWS, EMB), jnp.float32),
        pltpu.SemaphoreType.DMA,
    ],
)


def _gather_body(t0, t1, t2, idx_hbm, o0, o1, o2, idx_v, rows_v, sem):
  wid = lax.axis_index("s") * NC + lax.axis_index("c")
  per = 3072 // (NC * NS)
  b0 = wid * per
  pltpu.sync_copy(idx_hbm.at[pl.ds(b0, per)], idx_v)
  for t, o in ((t0, o0), (t1, o1), (t2, o2)):
    pltpu.async_copy(t.at[idx_v], rows_v, sem).wait()
    pltpu.sync_copy(rows_v, o.at[pl.ds(b0, per)])


_gather_kernel = pl.kernel(
    _gather_body,
    out_type=(jax.ShapeDtypeStruct((3072, EMB), jnp.float32),) * 3,
    mesh=_mesh,
    scratch_types=[
        pltpu.VMEM((96,), jnp.int32),
        pltpu.VMEM((96, EMB), jnp.float32),
        pltpu.SemaphoreType.DMA,
    ],
)


def _dense_block(ego_ref, side_ref, wg_ref, bg_ref, wb_ref, bb_ref,
                 new_ref, norm_ref):
  ego = ego_ref[...]
  side = side_ref[...]
  a = jnp.dot(ego + side, wg_ref[...],
              preferred_element_type=jnp.float32) + bg_ref[...]
  a = jnp.where(a > 0, a, 0.2 * a)
  b = jnp.dot(ego * side, wb_ref[...],
              preferred_element_type=jnp.float32) + bb_ref[...]
  b = jnp.where(b > 0, b, 0.2 * b)
  e = a + b
  new_ref[...] = e
  nrm = jnp.sqrt(jnp.sum(e * e, axis=1, keepdims=True))
  norm_ref[...] = e / jnp.maximum(nrm, 1e-12)


_BR = 10000

_dense_kernel = pl.pallas_call(
    _dense_block,
    grid=(NN // _BR,),
    in_specs=[
        pl.BlockSpec((_BR, EMB), lambda i: (i, 0)),
        pl.BlockSpec((_BR, EMB), lambda i: (i, 0)),
        pl.BlockSpec((EMB, EMB), lambda i: (0, 0)),
        pl.BlockSpec((1, EMB), lambda i: (0, 0)),
        pl.BlockSpec((EMB, EMB), lambda i: (0, 0)),
        pl.BlockSpec((1, EMB), lambda i: (0, 0)),
    ],
    out_specs=[
        pl.BlockSpec((_BR, EMB), lambda i: (i, 0)),
        pl.BlockSpec((_BR, EMB), lambda i: (i, 0)),
    ],
    out_shape=[
        jax.ShapeDtypeStruct((NN, EMB), jnp.float32),
        jax.ShapeDtypeStruct((NN, EMB), jnp.float32),
    ],
)


def kernel(users, pos_items, neg_items, edge_index, edge_weight,
           user_emb, item_emb,
           W_gc_0, b_gc_0, W_bi_0, b_bi_0,
           W_gc_1, b_gc_1, W_bi_1, b_bi_1):
  pad = EPAD - E
  src = jnp.concatenate([edge_index[0], jnp.zeros((pad,), jnp.int32)])
  dst = jnp.concatenate([edge_index[1], jnp.zeros((pad,), jnp.int32)])
  w = jnp.concatenate([edge_weight, jnp.zeros((pad,), jnp.float32)])

  ego0 = jnp.concatenate([user_emb, item_emb], axis=0)
  side1 = _side_kernel(src, dst, w, ego0)
  ego1, n1 = _dense_kernel(ego0, side1, W_gc_0, b_gc_0.reshape(1, EMB),
                           W_bi_0, b_bi_0.reshape(1, EMB))
  side2 = _side_kernel(src, dst, w, ego1)
  ego2, n2 = _dense_kernel(ego1, side2, W_gc_1, b_gc_1.reshape(1, EMB),
                           W_bi_1, b_bi_1.reshape(1, EMB))

  idx = jnp.concatenate([users, pos_items + N_USERS, neg_items + N_USERS])
  g0, g1, g2 = _gather_kernel(ego0, n1, n2, idx)
  out = jnp.concatenate([g0, g1, g2], axis=1)
  return out[:1024], out[1024:2048], out[2048:]


# speedup vs baseline: 4.1913x; 4.1913x over previous
import functools

import jax
import jax.numpy as jnp
from jax import lax
from jax.experimental import pallas as pl
from jax.experimental.pallas import tpu as pltpu
from jax.experimental.pallas import tpu_sc as plsc

N_USERS = 50000
N_ITEMS = 50000
NN = N_USERS + N_ITEMS
EMB = 32
E = 1600000

NC = 2
NS = 16
HALF = NN // NC
TRASH = 16 * NS
ACC_ROWS = 50688

KSTEP = 4
CHUNK = KSTEP * 128
NCHUNK = 196
EPT = NCHUNK * CHUNK
EPAD = EPT * NS
ZROWS = 32
ACC_PER_TILE = ACC_ROWS // NS
OUT_PER_TILE = 3120
OUT_REM = HALF - OUT_PER_TILE * NS

_mesh = plsc.VectorSubcoreMesh(
    core_axis_name="c", subcore_axis_name="s", num_cores=NC, num_subcores=NS)


def _side_body(src_hbm, dst_hbm, w_hbm, ego_hbm, out_hbm,
               src_vs, dst_v, w_v, idx_vs, rows_v, zbuf, acc, sem):
  cid = lax.axis_index("c")
  sid = lax.axis_index("s")
  base = cid * HALF

  def _zero_buf(i, _):
    zbuf[i, pl.ds(0, 16)] = jnp.zeros((16,), jnp.float32)
    zbuf[i, pl.ds(16, 16)] = jnp.zeros((16,), jnp.float32)
    return 0
  lax.fori_loop(0, ZROWS, _zero_buf, 0)

  def _zero_acc(i, _):
    pltpu.sync_copy(zbuf, acc.at[pl.ds(sid * ACC_PER_TILE + i * ZROWS, ZROWS)])
    return 0
  lax.fori_loop(0, ACC_PER_TILE // ZROWS, _zero_acc, 0)
  plsc.subcore_barrier()

  trash0 = HALF + sid * 16
  lane8 = lax.iota(jnp.int32, 16) & 7

  def _chunk(ci, _):
    off = sid * EPT + ci * CHUNK
    for j in range(KSTEP):
      pltpu.sync_copy(src_hbm.at[pl.ds(off + j * 128, 128)], src_vs[j])
    pltpu.sync_copy(dst_hbm.at[pl.ds(off, CHUNK)], dst_v)
    pltpu.sync_copy(w_hbm.at[pl.ds(off, CHUNK)], w_v)
    copies = [
        pltpu.async_copy(ego_hbm.at[src_vs[j]],
                         rows_v.at[pl.ds(j * 128, 128)], sem)
        for j in range(KSTEP)
    ]
    for c in copies:
      c.wait()

    for j in range(KSTEP):
      idx_v = idx_vs[j]

      def _blk(b, _, j=j, idx_v=idx_v):
        r0 = j * 128 + b * 16
        d = dst_v[pl.ds(r0, 16)]
        loc = d - base
        ok = (loc >= 0) & (loc < HALF)
        idx_v[pl.ds(b * 16, 16)] = jnp.where(ok, loc, trash0 + lane8)
        wv = w_v[pl.ds(r0, 16)]
        for jj in range(16):
          r = r0 + jj
          ws = wv[jj]
          rows_v[r, pl.ds(0, 16)] = rows_v[r, pl.ds(0, 16)] * ws
          rows_v[r, pl.ds(16, 16)] = rows_v[r, pl.ds(16, 16)] * ws
        return 0
      lax.fori_loop(0, 8, _blk, 0)

    for j in range(KSTEP):
      pltpu.sync_copy(rows_v.at[pl.ds(j * 128, 128)],
                      acc.at[idx_vs[j]], add=True)
    return 0
  lax.fori_loop(0, NCHUNK, _chunk, 0)
  plsc.subcore_barrier()

  pltpu.sync_copy(
      acc.at[pl.ds(sid * OUT_PER_TILE, OUT_PER_TILE)],
      out_hbm.at[pl.ds(base + sid * OUT_PER_TILE, OUT_PER_TILE)])

  @pl.when(sid == 0)
  def _rem():
    pltpu.sync_copy(
        acc.at[pl.ds(OUT_PER_TILE * NS, OUT_REM)],
        out_hbm.at[pl.ds(base + OUT_PER_TILE * NS, OUT_REM)])


_sc_params = pltpu.CompilerParams(use_tc_tiling_on_sc=False)

_side_kernel = pl.kernel(
    _side_body,
    out_type=jax.ShapeDtypeStruct((NN, EMB), jnp.float32),
    mesh=_mesh,
    compiler_params=_sc_params,
    scratch_types=[
        [pltpu.VMEM((128,), jnp.int32)] * KSTEP,
        pltpu.VMEM((CHUNK,), jnp.int32),
        pltpu.VMEM((CHUNK,), jnp.float32),
        [pltpu.VMEM((128,), jnp.int32)] * KSTEP,
        pltpu.VMEM((CHUNK, EMB), jnp.float32),
        pltpu.VMEM((ZROWS, EMB), jnp.float32),
        pltpu.VMEM_SHARED((ACC_ROWS, EMB), jnp.float32),
        pltpu.SemaphoreType.DMA,
    ],
)


def _gather_body(t0, t1, t2, idx_hbm, o0, o1, o2, idx_v, rows_v, sem):
  wid = lax.axis_index("s") * NC + lax.axis_index("c")
  per = 3072 // (NC * NS)
  b0 = wid * per
  pltpu.sync_copy(idx_hbm.at[pl.ds(b0, per)], idx_v)
  for t, o in ((t0, o0), (t1, o1), (t2, o2)):
    pltpu.async_copy(t.at[idx_v], rows_v, sem).wait()
    pltpu.sync_copy(rows_v, o.at[pl.ds(b0, per)])


_gather_kernel = pl.kernel(
    _gather_body,
    out_type=(jax.ShapeDtypeStruct((3072, EMB), jnp.float32),) * 3,
    mesh=_mesh,
    compiler_params=_sc_params,
    scratch_types=[
        pltpu.VMEM((96,), jnp.int32),
        pltpu.VMEM((96, EMB), jnp.float32),
        pltpu.SemaphoreType.DMA,
    ],
)


def _dense_block(ego_ref, side_ref, wg_ref, bg_ref, wb_ref, bb_ref,
                 new_ref, norm_ref):
  ego = ego_ref[...]
  side = side_ref[...]
  a = jnp.dot(ego + side, wg_ref[...],
              preferred_element_type=jnp.float32) + bg_ref[...]
  a = jnp.where(a > 0, a, 0.2 * a)
  b = jnp.dot(ego * side, wb_ref[...],
              preferred_element_type=jnp.float32) + bb_ref[...]
  b = jnp.where(b > 0, b, 0.2 * b)
  e = a + b
  new_ref[...] = e
  nrm = jnp.sqrt(jnp.sum(e * e, axis=1, keepdims=True))
  norm_ref[...] = e / jnp.maximum(nrm, 1e-12)


_BR = 10000

_dense_kernel = pl.pallas_call(
    _dense_block,
    grid=(NN // _BR,),
    in_specs=[
        pl.BlockSpec((_BR, EMB), lambda i: (i, 0)),
        pl.BlockSpec((_BR, EMB), lambda i: (i, 0)),
        pl.BlockSpec((EMB, EMB), lambda i: (0, 0)),
        pl.BlockSpec((1, EMB), lambda i: (0, 0)),
        pl.BlockSpec((EMB, EMB), lambda i: (0, 0)),
        pl.BlockSpec((1, EMB), lambda i: (0, 0)),
    ],
    out_specs=[
        pl.BlockSpec((_BR, EMB), lambda i: (i, 0)),
        pl.BlockSpec((_BR, EMB), lambda i: (i, 0)),
    ],
    out_shape=[
        jax.ShapeDtypeStruct((NN, EMB), jnp.float32),
        jax.ShapeDtypeStruct((NN, EMB), jnp.float32),
    ],
)


def kernel(users, pos_items, neg_items, edge_index, edge_weight,
           user_emb, item_emb,
           W_gc_0, b_gc_0, W_bi_0, b_bi_0,
           W_gc_1, b_gc_1, W_bi_1, b_bi_1):
  pad = EPAD - E
  src = jnp.concatenate([edge_index[0], jnp.zeros((pad,), jnp.int32)])
  dst = jnp.concatenate([edge_index[1], jnp.zeros((pad,), jnp.int32)])
  w = jnp.concatenate([edge_weight, jnp.zeros((pad,), jnp.float32)])

  ego0 = jnp.concatenate([user_emb, item_emb], axis=0)
  side1 = _side_kernel(src, dst, w, ego0)
  ego1, n1 = _dense_kernel(ego0, side1, W_gc_0, b_gc_0.reshape(1, EMB),
                           W_bi_0, b_bi_0.reshape(1, EMB))
  side2 = _side_kernel(src, dst, w, ego1)
  ego2, n2 = _dense_kernel(ego1, side2, W_gc_1, b_gc_1.reshape(1, EMB),
                           W_bi_1, b_bi_1.reshape(1, EMB))

  idx = jnp.concatenate([users, pos_items + N_USERS, neg_items + N_USERS])
  g0, g1, g2 = _gather_kernel(ego0, n1, n2, idx)
  out = jnp.concatenate([g0, g1, g2], axis=1)
  return out[:1024], out[1024:2048], out[2048:]


# speedup vs baseline: 4.8185x; 1.1496x over previous
import functools

import jax
import jax.numpy as jnp
from jax import lax
from jax.experimental import pallas as pl
from jax.experimental.pallas import tpu as pltpu
from jax.experimental.pallas import tpu_sc as plsc

N_USERS = 50000
N_ITEMS = 50000
NN = N_USERS + N_ITEMS
EMB = 32
HEMB = EMB // 2
E = 1600000

NC = 2
NS = 16
ACC_ROWS = 100352

KSTEP = 4
CHUNK = KSTEP * 128
NCHUNK = 196
EPT = NCHUNK * CHUNK
EPAD = EPT * NS
ZROWS = 32
ACC_PER_TILE = ACC_ROWS // NS
OUT_PER_TILE = 6240
OUT_REM = NN - OUT_PER_TILE * NS

_mesh = plsc.VectorSubcoreMesh(
    core_axis_name="c", subcore_axis_name="s", num_cores=NC, num_subcores=NS)


def _side_body(src_hbm, dst_hbm, w_hbm, ego_hbm, out_hbm,
               src_vs, dst_vs, w_v, idx_vs, rows_v, zbuf, acc, sem):
  cid = lax.axis_index("c")
  sid = lax.axis_index("s")
  srcoff = cid * NN

  def _zero_buf(i, _):
    zbuf[i, pl.ds(0, 16)] = jnp.zeros((16,), jnp.float32)
    return 0
  lax.fori_loop(0, ZROWS, _zero_buf, 0)

  def _zero_acc(i, _):
    pltpu.sync_copy(zbuf, acc.at[pl.ds(sid * ACC_PER_TILE + i * ZROWS, ZROWS)])
    return 0
  lax.fori_loop(0, ACC_PER_TILE // ZROWS, _zero_acc, 0)
  plsc.subcore_barrier()

  def _chunk(ci, _):
    off = sid * EPT + ci * CHUNK
    for j in range(KSTEP):
      pltpu.sync_copy(src_hbm.at[pl.ds(off + j * 128, 128)], src_vs[j])
      pltpu.sync_copy(dst_hbm.at[pl.ds(off + j * 128, 128)], dst_vs[j])
    pltpu.sync_copy(w_hbm.at[pl.ds(off, CHUNK)], w_v)

    for j in range(KSTEP):
      idx_v = idx_vs[j]

      def _idx(b, _, j=j, idx_v=idx_v):
        s = src_vs[j][pl.ds(b * 16, 16)]
        idx_v[pl.ds(b * 16, 16)] = s + srcoff
        return 0
      lax.fori_loop(0, 8, _idx, 0)

    copies = [
        pltpu.async_copy(ego_hbm.at[idx_vs[j]],
                         rows_v.at[pl.ds(j * 128, 128)], sem)
        for j in range(KSTEP)
    ]
    for c in copies:
      c.wait()

    def _blk(b, _):
      r0 = b * 16
      wv = w_v[pl.ds(r0, 16)]
      for jj in range(16):
        r = r0 + jj
        rows_v[r, pl.ds(0, 16)] = rows_v[r, pl.ds(0, 16)] * wv[jj]
      return 0
    lax.fori_loop(0, CHUNK // 16, _blk, 0)

    for j in range(KSTEP):
      pltpu.sync_copy(rows_v.at[pl.ds(j * 128, 128)],
                      acc.at[dst_vs[j]], add=True)
    return 0
  lax.fori_loop(0, NCHUNK, _chunk, 0)
  plsc.subcore_barrier()

  base = cid * NN
  pltpu.sync_copy(
      acc.at[pl.ds(sid * OUT_PER_TILE, OUT_PER_TILE)],
      out_hbm.at[pl.ds(base + sid * OUT_PER_TILE, OUT_PER_TILE)])

  @pl.when(sid == 0)
  def _rem():
    pltpu.sync_copy(
        acc.at[pl.ds(OUT_PER_TILE * NS, OUT_REM)],
        out_hbm.at[pl.ds(base + OUT_PER_TILE * NS, OUT_REM)])


_sc_params = pltpu.CompilerParams(use_tc_tiling_on_sc=False)

_side_kernel = pl.kernel(
    _side_body,
    out_type=jax.ShapeDtypeStruct((NC * NN, HEMB), jnp.float32),
    mesh=_mesh,
    compiler_params=_sc_params,
    scratch_types=[
        [pltpu.VMEM((128,), jnp.int32)] * KSTEP,
        [pltpu.VMEM((128,), jnp.int32)] * KSTEP,
        pltpu.VMEM((CHUNK,), jnp.float32),
        [pltpu.VMEM((128,), jnp.int32)] * KSTEP,
        pltpu.VMEM((CHUNK, HEMB), jnp.float32),
        pltpu.VMEM((ZROWS, HEMB), jnp.float32),
        pltpu.VMEM_SHARED((ACC_ROWS, HEMB), jnp.float32),
        pltpu.SemaphoreType.DMA,
    ],
)


def _gather_body(t0, t1, t2, idx_hbm, o0, o1, o2, idx_v, rows_v, sem):
  wid = lax.axis_index("s") * NC + lax.axis_index("c")
  per = 3072 // (NC * NS)
  b0 = wid * per
  pltpu.sync_copy(idx_hbm.at[pl.ds(b0, per)], idx_v)
  for t, o in ((t0, o0), (t1, o1), (t2, o2)):
    pltpu.async_copy(t.at[idx_v], rows_v, sem).wait()
    pltpu.sync_copy(rows_v, o.at[pl.ds(b0, per)])


_gather_kernel = pl.kernel(
    _gather_body,
    out_type=(jax.ShapeDtypeStruct((3072, EMB), jnp.float32),) * 3,
    mesh=_mesh,
    compiler_params=_sc_params,
    scratch_types=[
        pltpu.VMEM((96,), jnp.int32),
        pltpu.VMEM((96, EMB), jnp.float32),
        pltpu.SemaphoreType.DMA,
    ],
)


def _dense_block(ego_ref, sa_ref, sb_ref, wg_ref, bg_ref, wb_ref, bb_ref,
                 new_ref, norm_ref, spa_ref, spb_ref):
  ego = ego_ref[...]
  side = jnp.concatenate([sa_ref[...], sb_ref[...]], axis=1)
  a = jnp.dot(ego + side, wg_ref[...],
              preferred_element_type=jnp.float32) + bg_ref[...]
  a = jnp.where(a > 0, a, 0.2 * a)
  b = jnp.dot(ego * side, wb_ref[...],
              preferred_element_type=jnp.float32) + bb_ref[...]
  b = jnp.where(b > 0, b, 0.2 * b)
  e = a + b
  new_ref[...] = e
  nrm = jnp.sqrt(jnp.sum(e * e, axis=1, keepdims=True))
  norm_ref[...] = e / jnp.maximum(nrm, 1e-12)
  spa_ref[...] = e[:, :HEMB]
  spb_ref[...] = e[:, HEMB:]


_BR = 5000

_dense_kernel = pl.pallas_call(
    _dense_block,
    grid=(NN // _BR,),
    in_specs=[
        pl.BlockSpec((_BR, EMB), lambda i: (i, 0)),
        pl.BlockSpec((_BR, HEMB), lambda i: (i, 0)),
        pl.BlockSpec((_BR, HEMB), lambda i: (i + NN // _BR, 0)),
        pl.BlockSpec((EMB, EMB), lambda i: (0, 0)),
        pl.BlockSpec((1, EMB), lambda i: (0, 0)),
        pl.BlockSpec((EMB, EMB), lambda i: (0, 0)),
        pl.BlockSpec((1, EMB), lambda i: (0, 0)),
    ],
    out_specs=[
        pl.BlockSpec((_BR, EMB), lambda i: (i, 0)),
        pl.BlockSpec((_BR, EMB), lambda i: (i, 0)),
        pl.BlockSpec((_BR, HEMB), lambda i: (i, 0)),
        pl.BlockSpec((_BR, HEMB), lambda i: (i, 0)),
    ],
    out_shape=[
        jax.ShapeDtypeStruct((NN, EMB), jnp.float32),
        jax.ShapeDtypeStruct((NN, EMB), jnp.float32),
        jax.ShapeDtypeStruct((NN, HEMB), jnp.float32),
        jax.ShapeDtypeStruct((NN, HEMB), jnp.float32),
    ],
)


def _dense(ego, side_flat, Wg, bg, Wb, bb):
  new, norm, spa, spb = _dense_kernel(
      ego, side_flat, side_flat, Wg, bg.reshape(1, EMB),
      Wb, bb.reshape(1, EMB))
  return new, norm, jnp.concatenate([spa, spb], axis=0)


def kernel(users, pos_items, neg_items, edge_index, edge_weight,
           user_emb, item_emb,
           W_gc_0, b_gc_0, W_bi_0, b_bi_0,
           W_gc_1, b_gc_1, W_bi_1, b_bi_1):
  pad = EPAD - E
  src = jnp.concatenate([edge_index[0], jnp.zeros((pad,), jnp.int32)])
  dst = jnp.concatenate([edge_index[1], jnp.zeros((pad,), jnp.int32)])
  w = jnp.concatenate([edge_weight, jnp.zeros((pad,), jnp.float32)])

  ego0 = jnp.concatenate([user_emb, item_emb], axis=0)
  split0 = jnp.concatenate([ego0[:, :HEMB], ego0[:, HEMB:]], axis=0)
  side1 = _side_kernel(src, dst, w, split0)
  ego1, n1, split1 = _dense(ego0, side1, W_gc_0, b_gc_0, W_bi_0, b_bi_0)
  side2 = _side_kernel(src, dst, w, split1)
  ego2, n2, split2 = _dense(ego1, side2, W_gc_1, b_gc_1, W_bi_1, b_bi_1)

  idx = jnp.concatenate([users, pos_items + N_USERS, neg_items + N_USERS])
  g0, g1, g2 = _gather_kernel(ego0, n1, n2, idx)
  out = jnp.concatenate([g0, g1, g2], axis=1)
  return out[:1024], out[1024:2048], out[2048:]


# speedup vs baseline: 5.5178x; 1.1451x over previous
import functools

import jax
import jax.numpy as jnp
from jax import lax
from jax.experimental import pallas as pl
from jax.experimental.pallas import tpu as pltpu
from jax.experimental.pallas import tpu_sc as plsc

N_USERS = 50000
N_ITEMS = 50000
NN = N_USERS + N_ITEMS
EMB = 32
HEMB = EMB // 2
E = 1600000

NC = 2
NS = 16
ACC_ROWS = 100352

KSTEP = 4
CHUNK = KSTEP * 128
NCHUNK = 196
EPT = NCHUNK * CHUNK
EPAD = EPT * NS
ZROWS = 32
ACC_PER_TILE = ACC_ROWS // NS
OUT_PER_TILE = 6240
OUT_REM = NN - OUT_PER_TILE * NS

_mesh = plsc.VectorSubcoreMesh(
    core_axis_name="c", subcore_axis_name="s", num_cores=NC, num_subcores=NS)


def _side_body(src_hbm, dst_hbm, w_hbm, ego_hbm, out_hbm,
               src_vs, dst_vs, w_v, idx_vs, rows_v, zbuf, acc, sems):
  cid = lax.axis_index("c")
  sid = lax.axis_index("s")
  srcoff = cid * NN

  def _zero_buf(i, _):
    zbuf[i, pl.ds(0, 16)] = jnp.zeros((16,), jnp.float32)
    return 0
  lax.fori_loop(0, ZROWS, _zero_buf, 0)

  def _zero_acc(i, _):
    pltpu.sync_copy(zbuf, acc.at[pl.ds(sid * ACC_PER_TILE + i * ZROWS, ZROWS)])
    return 0
  lax.fori_loop(0, ACC_PER_TILE // ZROWS, _zero_acc, 0)
  plsc.subcore_barrier()

  def _load_issue(ci, half):
    off = sid * EPT + ci * CHUNK
    for j in range(KSTEP):
      jj = half * KSTEP + j
      pltpu.sync_copy(src_hbm.at[pl.ds(off + j * 128, 128)], src_vs[jj])
      pltpu.sync_copy(dst_hbm.at[pl.ds(off + j * 128, 128)], dst_vs[jj])
    pltpu.sync_copy(w_hbm.at[pl.ds(off, CHUNK)],
                    w_v.at[pl.ds(half * CHUNK, CHUNK)])
    for j in range(KSTEP):
      jj = half * KSTEP + j
      idx_v = idx_vs[jj]

      def _idx(b, _, jj=jj, idx_v=idx_v):
        s = src_vs[jj][pl.ds(b * 16, 16)]
        idx_v[pl.ds(b * 16, 16)] = s + srcoff
        return 0
      lax.fori_loop(0, 8, _idx, 0)
    return [
        pltpu.async_copy(ego_hbm.at[idx_vs[half * KSTEP + j]],
                         rows_v.at[pl.ds(half * CHUNK + j * 128, 128)],
                         sems[half])
        for j in range(KSTEP)
    ]

  def _proc(half):
    def _blk(b, _):
      r0 = half * CHUNK + b * 16
      wv = w_v[pl.ds(r0, 16)]
      for jj in range(16):
        r = r0 + jj
        rows_v[r, pl.ds(0, 16)] = rows_v[r, pl.ds(0, 16)] * wv[jj]
      return 0
    lax.fori_loop(0, CHUNK // 16, _blk, 0)
    for j in range(KSTEP):
      pltpu.sync_copy(rows_v.at[pl.ds(half * CHUNK + j * 128, 128)],
                      acc.at[dst_vs[half * KSTEP + j]], add=True)

  def _pair(i, _):
    copies_a = _load_issue(2 * i, 0)
    copies_b = _load_issue(2 * i + 1, 1)
    for c in copies_a:
      c.wait()
    _proc(0)
    for c in copies_b:
      c.wait()
    _proc(1)
    return 0
  lax.fori_loop(0, NCHUNK // 2, _pair, 0)
  plsc.subcore_barrier()

  base = cid * NN
  pltpu.sync_copy(
      acc.at[pl.ds(sid * OUT_PER_TILE, OUT_PER_TILE)],
      out_hbm.at[pl.ds(base + sid * OUT_PER_TILE, OUT_PER_TILE)])

  @pl.when(sid == 0)
  def _rem():
    pltpu.sync_copy(
        acc.at[pl.ds(OUT_PER_TILE * NS, OUT_REM)],
        out_hbm.at[pl.ds(base + OUT_PER_TILE * NS, OUT_REM)])


_sc_params = pltpu.CompilerParams(use_tc_tiling_on_sc=False)

_side_kernel = pl.kernel(
    _side_body,
    out_type=jax.ShapeDtypeStruct((NC * NN, HEMB), jnp.float32),
    mesh=_mesh,
    compiler_params=_sc_params,
    scratch_types=[
        [pltpu.VMEM((128,), jnp.int32)] * (2 * KSTEP),
        [pltpu.VMEM((128,), jnp.int32)] * (2 * KSTEP),
        pltpu.VMEM((2 * CHUNK,), jnp.float32),
        [pltpu.VMEM((128,), jnp.int32)] * (2 * KSTEP),
        pltpu.VMEM((2 * CHUNK, HEMB), jnp.float32),
        pltpu.VMEM((ZROWS, HEMB), jnp.float32),
        pltpu.VMEM_SHARED((ACC_ROWS, HEMB), jnp.float32),
        [pltpu.SemaphoreType.DMA] * 2,
    ],
)


def _gather_body(t0, t1, t2, idx_hbm, o0, o1, o2, idx_v, rows_v, sem):
  wid = lax.axis_index("s") * NC + lax.axis_index("c")
  per = 3072 // (NC * NS)
  b0 = wid * per
  pltpu.sync_copy(idx_hbm.at[pl.ds(b0, per)], idx_v)
  for t, o in ((t0, o0), (t1, o1), (t2, o2)):
    pltpu.async_copy(t.at[idx_v], rows_v, sem).wait()
    pltpu.sync_copy(rows_v, o.at[pl.ds(b0, per)])


_gather_kernel = pl.kernel(
    _gather_body,
    out_type=(jax.ShapeDtypeStruct((3072, EMB), jnp.float32),) * 3,
    mesh=_mesh,
    compiler_params=_sc_params,
    scratch_types=[
        pltpu.VMEM((96,), jnp.int32),
        pltpu.VMEM((96, EMB), jnp.float32),
        pltpu.SemaphoreType.DMA,
    ],
)


def _dense_block(ego_ref, sa_ref, sb_ref, wg_ref, bg_ref, wb_ref, bb_ref,
                 new_ref, norm_ref, spa_ref, spb_ref):
  ego = ego_ref[...]
  side = jnp.concatenate([sa_ref[...], sb_ref[...]], axis=1)
  a = jnp.dot(ego + side, wg_ref[...],
              preferred_element_type=jnp.float32) + bg_ref[...]
  a = jnp.where(a > 0, a, 0.2 * a)
  b = jnp.dot(ego * side, wb_ref[...],
              preferred_element_type=jnp.float32) + bb_ref[...]
  b = jnp.where(b > 0, b, 0.2 * b)
  e = a + b
  new_ref[...] = e
  nrm = jnp.sqrt(jnp.sum(e * e, axis=1, keepdims=True))
  norm_ref[...] = e / jnp.maximum(nrm, 1e-12)
  spa_ref[...] = e[:, :HEMB]
  spb_ref[...] = e[:, HEMB:]


_BR = 5000

_dense_kernel = pl.pallas_call(
    _dense_block,
    grid=(NN // _BR,),
    in_specs=[
        pl.BlockSpec((_BR, EMB), lambda i: (i, 0)),
        pl.BlockSpec((_BR, HEMB), lambda i: (i, 0)),
        pl.BlockSpec((_BR, HEMB), lambda i: (i + NN // _BR, 0)),
        pl.BlockSpec((EMB, EMB), lambda i: (0, 0)),
        pl.BlockSpec((1, EMB), lambda i: (0, 0)),
        pl.BlockSpec((EMB, EMB), lambda i: (0, 0)),
        pl.BlockSpec((1, EMB), lambda i: (0, 0)),
    ],
    out_specs=[
        pl.BlockSpec((_BR, EMB), lambda i: (i, 0)),
        pl.BlockSpec((_BR, EMB), lambda i: (i, 0)),
        pl.BlockSpec((_BR, HEMB), lambda i: (i, 0)),
        pl.BlockSpec((_BR, HEMB), lambda i: (i, 0)),
    ],
    out_shape=[
        jax.ShapeDtypeStruct((NN, EMB), jnp.float32),
        jax.ShapeDtypeStruct((NN, EMB), jnp.float32),
        jax.ShapeDtypeStruct((NN, HEMB), jnp.float32),
        jax.ShapeDtypeStruct((NN, HEMB), jnp.float32),
    ],
)


def _dense(ego, side_flat, Wg, bg, Wb, bb):
  new, norm, spa, spb = _dense_kernel(
      ego, side_flat, side_flat, Wg, bg.reshape(1, EMB),
      Wb, bb.reshape(1, EMB))
  return new, norm, jnp.concatenate([spa, spb], axis=0)


def kernel(users, pos_items, neg_items, edge_index, edge_weight,
           user_emb, item_emb,
           W_gc_0, b_gc_0, W_bi_0, b_bi_0,
           W_gc_1, b_gc_1, W_bi_1, b_bi_1):
  pad = EPAD - E
  src = jnp.concatenate([edge_index[0], jnp.zeros((pad,), jnp.int32)])
  dst = jnp.concatenate([edge_index[1], jnp.zeros((pad,), jnp.int32)])
  w = jnp.concatenate([edge_weight, jnp.zeros((pad,), jnp.float32)])

  ego0 = jnp.concatenate([user_emb, item_emb], axis=0)
  split0 = jnp.concatenate([ego0[:, :HEMB], ego0[:, HEMB:]], axis=0)
  side1 = _side_kernel(src, dst, w, split0)
  ego1, n1, split1 = _dense(ego0, side1, W_gc_0, b_gc_0, W_bi_0, b_bi_0)
  side2 = _side_kernel(src, dst, w, split1)
  ego2, n2, split2 = _dense(ego1, side2, W_gc_1, b_gc_1, W_bi_1, b_bi_1)

  idx = jnp.concatenate([users, pos_items + N_USERS, neg_items + N_USERS])
  g0, g1, g2 = _gather_kernel(ego0, n1, n2, idx)
  out = jnp.concatenate([g0, g1, g2], axis=1)
  return out[:1024], out[1024:2048], out[2048:]
